# baseline (speedup 1.0000x reference)
import functools

import jax
import jax.numpy as jnp
from jax import lax
from jax.experimental import pallas as pl
from jax.experimental.pallas import tpu as pltpu
from jax.experimental.pallas import tpu_sc as plsc

B = 16384
D = 16
H = 32
NC, NS = 2, 16
NW = NC * NS
BPW = B // NW
GC = 128
NCHUNK = BPW // GC
L = 16
NVEC = BPW // L


def _bf(v):
    u = plsc.bitcast(v, jnp.uint32)
    r = (u + jnp.uint32(0x7FFF) + ((u >> 16) & jnp.uint32(1))) & jnp.uint32(
        0xFFFF0000)
    return plsc.bitcast(r, jnp.float32)


def _tree_sum(terms):
    while len(terms) > 1:
        nxt = [terms[i] + terms[i + 1] for i in range(0, len(terms) - 1, 2)]
        if len(terms) % 2:
            nxt.append(terms[-1])
        terms = nxt
    return terms[0]


@functools.cache
def _make_sc_kernel():
    mesh = plsc.VectorSubcoreMesh(
        core_axis_name="c", subcore_axis_name="s", num_cores=NC, num_subcores=NS
    )

    @functools.partial(
        pl.kernel,
        out_type=jax.ShapeDtypeStruct((B,), jnp.float32),
        mesh=mesh,
        scratch_types=[
            pltpu.VMEM((BPW, 2), jnp.float32),
            pltpu.VMEM((NCHUNK, GC), jnp.int32),
            pltpu.VMEM((BPW,), jnp.float32),
            pltpu.VMEM((BPW, D), jnp.float32),
            pltpu.VMEM((BPW,), jnp.float32),
            pltpu.VMEM((H, D + 1), jnp.float32),
            pltpu.VMEM((H,), jnp.float32),
            pltpu.VMEM((H, H), jnp.float32),
            pltpu.VMEM((H,), jnp.float32),
            pltpu.VMEM((1, H + 1), jnp.float32),
            pltpu.VMEM((H, D + 1, L), jnp.float32),
            pltpu.VMEM((H, L), jnp.float32),
            pltpu.VMEM((H, H, L), jnp.float32),
            pltpu.VMEM((H, L), jnp.float32),
            pltpu.VMEM((H, L), jnp.float32),
            pltpu.VMEM((L,), jnp.float32),
            pltpu.VMEM((NVEC, H, L), jnp.float32),
            pltpu.VMEM((NVEC, H, L), jnp.float32),
            pltpu.SemaphoreType.DMA,
        ],
        compiler_params=pltpu.CompilerParams(
            use_tc_tiling_on_sc=False, needs_layout_passes=False
        ),
    )
    def sc_kernel(x_hbm, table_hbm, w1_hbm, b1_hbm, w2_hbm, b2_hbm, w3_hbm,
                  out_hbm, x_v, idx_v, year_v, rows_v, out_v,
                  w1_v, b1_v, w2_v, b2_v, w3_v,
                  w1s_v, b1s_v, w2s_v, b2s_v, w3s_v, b3s_v, h1_v, h2_v, sem):
        wid = lax.axis_index("s") * NC + lax.axis_index("c")
        base = wid * BPW

        pltpu.sync_copy(x_hbm.at[pl.ds(base, BPW)], x_v)
        pltpu.sync_copy(w1_hbm, w1_v)
        pltpu.sync_copy(b1_hbm, b1_v)
        pltpu.sync_copy(w2_hbm, w2_v)
        pltpu.sync_copy(b2_hbm, b2_v)
        pltpu.sync_copy(w3_hbm, w3_v)

        iota = lax.iota(jnp.int32, L)
        zeros = jnp.zeros((L,), jnp.int32)
        ones = jnp.ones((L,), jnp.int32)

        @plsc.parallel_loop(0, NVEC, step=1)
        def extract(c):
            rows = c * L + iota
            cf = plsc.load_gather(x_v, [rows, zeros])
            yf = plsc.load_gather(x_v, [rows, ones])
            idx_v[c // (GC // L), pl.ds((c % (GC // L)) * L, L)] = (
                cf.astype(jnp.int32))
            year_v[pl.ds(c * L, L)] = yf

        @plsc.parallel_loop(0, H, step=1)
        def build(j):
            js = jnp.full((L,), j, jnp.int32)
            for k in range(D + 1):
                w1s_v[j, k] = _bf(plsc.load_gather(
                    w1_v, [js, jnp.full((L,), k, jnp.int32)]))
            for k in range(H):
                w2s_v[j, k] = _bf(plsc.load_gather(
                    w2_v, [js, jnp.full((L,), k, jnp.int32)]))
            b1s_v[j] = plsc.load_gather(b1_v, [js])
            b2s_v[j] = plsc.load_gather(b2_v, [js])
            w3s_v[j] = _bf(plsc.load_gather(w3_v, [zeros, js]))

        b3s_v[...] = plsc.load_gather(w3_v, [zeros, jnp.full((L,), H, jnp.int32)])

        copies = [
            pltpu.async_copy(
                table_hbm.at[idx_v.at[j]], rows_v.at[pl.ds(j * GC, GC)], sem
            )
            for j in range(NCHUNK)
        ]
        for c in copies:
            c.wait()

        JB = 8

        @plsc.parallel_loop(0, NVEC, step=1)
        def mlp1(c):
            rows = c * L + iota
            year = _bf(year_v[pl.ds(c * L, L)])
            for jb in range(0, H, JB):
                acc = [b1s_v[jb + j] + year * w1s_v[jb + j, 0]
                       for j in range(JB)]
                for k in range(D):
                    ek = _bf(plsc.load_gather(
                        rows_v, [rows, jnp.full((L,), k, jnp.int32)]))
                    acc = [acc[j] + ek * w1s_v[jb + j, k + 1]
                           for j in range(JB)]
                for j in range(JB):
                    h1_v[c, jb + j] = _bf(jnp.maximum(acc[j], 0.0))

        @plsc.parallel_loop(0, NVEC, step=1)
        def mlp2(c):
            h2 = []
            for jb in range(0, H, JB):
                acc = [b2s_v[jb + j] for j in range(JB)]
                for k in range(H):
                    hk = h1_v[c, k]
                    acc = [acc[j] + hk * w2s_v[jb + j, k] for j in range(JB)]
                h2 += [_bf(jnp.maximum(acc[j], 0.0)) for j in range(JB)]
            terms = [h2[k] * w3s_v[k] for k in range(H)]
            terms.append(b3s_v[...])
            out_v[pl.ds(c * L, L)] = _tree_sum(terms)

        pltpu.sync_copy(out_v, out_hbm.at[pl.ds(base, BPW)])

    return sc_kernel


def kernel(x, embed, W1, b1, W2, b2, W3, b3):
    w3c = jnp.concatenate([W3, b3.reshape(1, 1)], axis=1)
    out = _make_sc_kernel()(x, embed, W1, b1, W2, b2, w3c)
    return out.reshape(B, 1)


# speedup vs baseline: 2.0246x; 2.0246x over previous
import functools

import jax
import jax.numpy as jnp
from jax import lax
from jax.experimental import pallas as pl
from jax.experimental.pallas import tpu as pltpu
from jax.experimental.pallas import tpu_sc as plsc

B = 16384
D = 16
H = 32
NC, NS = 2, 16
NW = NC * NS
BPW = B // NW
GC = 128
NCHUNK = BPW // GC
L = 16
NVEC = BPW // L
BK = 4096


@functools.cache
def _make_sc_gather():
    mesh = plsc.VectorSubcoreMesh(
        core_axis_name="c", subcore_axis_name="s", num_cores=NC, num_subcores=NS
    )

    @functools.partial(
        pl.kernel,
        out_type=jax.ShapeDtypeStruct((B, D), jnp.float32),
        mesh=mesh,
        scratch_types=[
            pltpu.VMEM((BPW, 2), jnp.float32),
            pltpu.VMEM((NCHUNK, GC), jnp.int32),
            pltpu.VMEM((BPW, D), jnp.float32),
            pltpu.SemaphoreType.DMA,
        ],
        compiler_params=pltpu.CompilerParams(
            use_tc_tiling_on_sc=False, needs_layout_passes=False
        ),
    )
    def sc_gather(x_hbm, table_hbm, out_hbm, x_v, idx_v, rows_v, sem):
        wid = lax.axis_index("s") * NC + lax.axis_index("c")
        base = wid * BPW
        pltpu.sync_copy(x_hbm.at[pl.ds(base, BPW)], x_v)

        iota = lax.iota(jnp.int32, L)
        zeros = jnp.zeros((L,), jnp.int32)

        @plsc.parallel_loop(0, NVEC, step=1)
        def extract(c):
            rows = c * L + iota
            cf = plsc.load_gather(x_v, [rows, zeros])
            idx_v[c // (GC // L), pl.ds((c % (GC // L)) * L, L)] = (
                cf.astype(jnp.int32))

        copies = [
            pltpu.async_copy(
                table_hbm.at[idx_v.at[j]], rows_v.at[pl.ds(j * GC, GC)], sem
            )
            for j in range(NCHUNK)
        ]
        for c in copies:
            c.wait()
        pltpu.sync_copy(rows_v, out_hbm.at[pl.ds(base, BPW)])

    return sc_gather


def _mlp_body(x_ref, e_ref, w1y_ref, w1e_ref, b1_ref, w2_ref, b2_ref,
              w3_ref, b3_ref, out_ref):
    e = e_ref[...]
    year = x_ref[:, 1:2]
    h1 = lax.dot_general(e, w1e_ref[...], (((1,), (1,)), ((), ())),
                         preferred_element_type=jnp.float32)
    h1y = lax.dot_general(year, w1y_ref[...], (((1,), (0,)), ((), ())),
                          preferred_element_type=jnp.float32)
    h1 = jnp.maximum(h1 + h1y + b1_ref[...], 0.0)
    h2 = lax.dot_general(h1, w2_ref[...], (((1,), (1,)), ((), ())),
                         preferred_element_type=jnp.float32)
    h2 = jnp.maximum(h2 + b2_ref[...], 0.0)
    w3b = jnp.broadcast_to(w3_ref[...], (8, H))
    out = lax.dot_general(h2, w3b, (((1,), (1,)), ((), ())),
                          preferred_element_type=jnp.float32)
    out_ref[...] = out[:, :1] + b3_ref[0]


def _mlp(x, e, w1y, w1e, b1, w2, b2, w3, b3):
    full = lambda s: pl.BlockSpec(s, lambda i: (0, 0))
    return pl.pallas_call(
        _mlp_body,
        grid=(B // BK,),
        in_specs=[
            pl.BlockSpec((BK, 2), lambda i: (i, 0)),
            pl.BlockSpec((BK, D), lambda i: (i, 0)),
            full((1, H)),
            full((H, D)),
            full((1, H)),
            full((H, H)),
            full((1, H)),
            full((1, H)),
            pl.BlockSpec(memory_space=pltpu.SMEM),
        ],
        out_specs=pl.BlockSpec((BK, 1), lambda i: (i, 0)),
        out_shape=jax.ShapeDtypeStruct((B, 1), jnp.float32),
    )(x, e, w1y, w1e, b1, w2, b2, w3, b3)


def kernel(x, embed, W1, b1, W2, b2, W3, b3):
    e = _make_sc_gather()(x, embed)
    w1y = W1[:, 0].reshape(1, H)
    w1e = W1[:, 1:]
    return _mlp(x, e, w1y, w1e, b1.reshape(1, H), W2, b2.reshape(1, H),
                W3, b3)


# speedup vs baseline: 2.1650x; 1.0694x over previous
import functools

import jax
import jax.numpy as jnp
from jax import lax
from jax.experimental import pallas as pl
from jax.experimental.pallas import tpu as pltpu
from jax.experimental.pallas import tpu_sc as plsc

B = 16384
D = 16
H = 32
NC, NS = 2, 16
NW = NC * NS
BPW = B // NW
GC = 128
NCHUNK = BPW // GC
L = 16
NVEC = BPW // L
BK = 4096


@functools.cache
def _make_sc_gather():
    mesh = plsc.VectorSubcoreMesh(
        core_axis_name="c", subcore_axis_name="s", num_cores=NC, num_subcores=NS
    )

    @functools.partial(
        pl.kernel,
        out_type=jax.ShapeDtypeStruct((B, D), jnp.float32),
        mesh=mesh,
        scratch_types=[
            pltpu.VMEM((BPW * 2,), jnp.float32),
            pltpu.VMEM((NCHUNK, GC), jnp.int32),
            pltpu.VMEM((BPW, D), jnp.float32),
            pltpu.SemaphoreType.DMA,
        ],
        compiler_params=pltpu.CompilerParams(
            use_tc_tiling_on_sc=False, needs_layout_passes=False
        ),
    )
    def sc_gather(x_hbm, table_hbm, out_hbm, x_v, idx_v, rows_v, sem):
        wid = lax.axis_index("s") * NC + lax.axis_index("c")
        base = wid * BPW
        pltpu.sync_copy(x_hbm.at[pl.ds(base * 2, BPW * 2)], x_v)

        iota2 = lax.iota(jnp.int32, L) * 2

        @plsc.parallel_loop(0, NVEC, step=1)
        def extract(c):
            cf = plsc.load_gather(x_v, [c * (2 * L) + iota2])
            idx_v[c // (GC // L), pl.ds((c % (GC // L)) * L, L)] = (
                cf.astype(jnp.int32))

        copies = [
            pltpu.async_copy(
                table_hbm.at[idx_v.at[j]], rows_v.at[pl.ds(j * GC, GC)], sem
            )
            for j in range(NCHUNK)
        ]
        for c in copies:
            c.wait()
        pltpu.sync_copy(rows_v, out_hbm.at[pl.ds(base, BPW)])

    return sc_gather


def _mlp_body(x_ref, e_ref, w1y_ref, w1e_ref, b1_ref, w2_ref, b2_ref,
              w3_ref, b3_ref, out_ref):
    e = e_ref[...]
    year = x_ref[:, 1:2]
    h1 = lax.dot_general(e, w1e_ref[...], (((1,), (1,)), ((), ())),
                         preferred_element_type=jnp.float32)
    h1y = lax.dot_general(year, w1y_ref[...], (((1,), (0,)), ((), ())),
                          preferred_element_type=jnp.float32)
    h1 = jnp.maximum(h1 + h1y + b1_ref[...], 0.0)
    h2 = lax.dot_general(h1, w2_ref[...], (((1,), (1,)), ((), ())),
                         preferred_element_type=jnp.float32)
    h2 = jnp.maximum(h2 + b2_ref[...], 0.0)
    w3b = jnp.broadcast_to(w3_ref[...], (8, H))
    out = lax.dot_general(h2, w3b, (((1,), (1,)), ((), ())),
                          preferred_element_type=jnp.float32)
    out_ref[...] = out[:, :1] + b3_ref[0]


def _mlp(x, e, w1y, w1e, b1, w2, b2, w3, b3):
    full = lambda s: pl.BlockSpec(s, lambda i: (0, 0))
    return pl.pallas_call(
        _mlp_body,
        grid=(B // BK,),
        in_specs=[
            pl.BlockSpec((BK, 2), lambda i: (i, 0)),
            pl.BlockSpec((BK, D), lambda i: (i, 0)),
            full((1, H)),
            full((H, D)),
            full((1, H)),
            full((H, H)),
            full((1, H)),
            full((1, H)),
            pl.BlockSpec(memory_space=pltpu.SMEM),
        ],
        out_specs=pl.BlockSpec((BK, 1), lambda i: (i, 0)),
        out_shape=jax.ShapeDtypeStruct((B, 1), jnp.float32),
    )(x, e, w1y, w1e, b1, w2, b2, w3, b3)


def kernel(x, embed, W1, b1, W2, b2, W3, b3):
    e = _make_sc_gather()(x.reshape(B * 2), embed)
    w1y = W1[:, 0].reshape(1, H)
    w1e = W1[:, 1:]
    return _mlp(x, e, w1y, w1e, b1.reshape(1, H), W2, b2.reshape(1, H),
                W3, b3)


# speedup vs baseline: 2.3231x; 1.0730x over previous
import functools

import jax
import jax.numpy as jnp
from jax import lax
from jax.scipy.linalg import block_diag
from jax.experimental import pallas as pl
from jax.experimental.pallas import tpu as pltpu
from jax.experimental.pallas import tpu_sc as plsc

B = 16384
D = 16
H = 32
NC, NS = 2, 16
NW = NC * NS
BPW = B // NW
L = 16
NVEC = BPW // L
EPW = BPW * D
NER = EPW // 128
G = B // 8
BG = 1024


@functools.cache
def _make_sc_gather():
    mesh = plsc.VectorSubcoreMesh(
        core_axis_name="c", subcore_axis_name="s", num_cores=NC, num_subcores=NS
    )

    @functools.partial(
        pl.kernel,
        out_type=jax.ShapeDtypeStruct((B * D,), jnp.float32),
        mesh=mesh,
        scratch_types=[
            pltpu.VMEM((BPW * 2,), jnp.float32),
            pltpu.VMEM((NER, 128), jnp.int32),
            pltpu.VMEM((NER, 128), jnp.float32),
            pltpu.VMEM((EPW,), jnp.float32),
            pltpu.SemaphoreType.DMA,
        ],
        compiler_params=pltpu.CompilerParams(
            use_tc_tiling_on_sc=False, needs_layout_passes=False
        ),
    )
    def sc_gather(x_hbm, table_hbm, out_hbm, x_v, eidx_v, rows2_v, out_v, sem):
        wid = lax.axis_index("s") * NC + lax.axis_index("c")
        base = wid * BPW
        pltpu.sync_copy(x_hbm.at[pl.ds(base * 2, BPW * 2)], x_v)

        iota = lax.iota(jnp.int32, L)
        iota2 = iota * 2

        @plsc.parallel_loop(0, NVEC, step=1)
        def extract(c):
            cf = plsc.load_gather(x_v, [c * (2 * L) + iota2])
            ebase = cf.astype(jnp.int32) * D
            for k in range(D):
                p = k * L
                eidx_v[c * 2 + p // 128, pl.ds(p % 128, L)] = ebase + k

        copies = [
            pltpu.async_copy(
                table_hbm.at[eidx_v.at[j]], rows2_v.at[j], sem
            )
            for j in range(NER)
        ]
        for c in copies:
            c.wait()

        @plsc.parallel_loop(0, NVEC, step=1)
        def transpose(c):
            dst = (c * L + iota) * D
            for k in range(D):
                p = k * L
                vk = rows2_v[c * 2 + p // 128, pl.ds(p % 128, L)]
                plsc.store_scatter(out_v, [dst + k], vk)

        pltpu.sync_copy(out_v, out_hbm.at[pl.ds(wid * EPW, EPW)])

    return sc_gather


def _mlp_body(e_ref, y_ref, w1_ref, wy_ref, b1_ref, w2_ref, b2_ref,
              w3_ref, b3_ref, out_ref):
    h1 = lax.dot_general(e_ref[...], w1_ref[...], (((1,), (0,)), ((), ())),
                         preferred_element_type=jnp.float32)
    h1y = lax.dot_general(y_ref[...], wy_ref[...], (((1,), (0,)), ((), ())),
                          preferred_element_type=jnp.float32)
    h1 = jnp.maximum(h1 + h1y + b1_ref[...], 0.0)
    h2 = lax.dot_general(h1, w2_ref[...], (((1,), (0,)), ((), ())),
                         preferred_element_type=jnp.float32)
    h2 = jnp.maximum(h2 + b2_ref[...], 0.0)
    out = lax.dot_general(h2, w3_ref[...], (((1,), (0,)), ((), ())),
                          preferred_element_type=jnp.float32)
    out_ref[...] = out + b3_ref[0]


def _mlp(e2, y2, w1p, wyp, b1p, w2p, b2p, w3p, b3):
    full = lambda s: pl.BlockSpec(s, lambda i: (0, 0))
    return pl.pallas_call(
        _mlp_body,
        grid=(G // BG,),
        in_specs=[
            pl.BlockSpec((BG, 128), lambda i: (i, 0)),
            pl.BlockSpec((BG, 8), lambda i: (i, 0)),
            full((128, 256)),
            full((8, 256)),
            full((1, 256)),
            full((256, 256)),
            full((1, 256)),
            full((256, 8)),
            pl.BlockSpec(memory_space=pltpu.SMEM),
        ],
        out_specs=pl.BlockSpec((BG, 8), lambda i: (i, 0)),
        out_shape=jax.ShapeDtypeStruct((G, 8), jnp.float32),
    )(e2, y2, w1p, wyp, b1p, w2p, b2p, w3p, b3)


def kernel(x, embed, W1, b1, W2, b2, W3, b3):
    e_flat = _make_sc_gather()(x.reshape(B * 2), embed.reshape(-1))
    e2 = e_flat.reshape(G, 8 * D)
    y2 = x[:, 1].reshape(G, 8)
    w1e_t = W1[:, 1:].T
    wy_t = W1[:, 0].reshape(1, H)
    w1p = block_diag(*([w1e_t] * 8))
    wyp = block_diag(*([wy_t] * 8))
    w2p = block_diag(*([W2.T] * 8))
    w3p = block_diag(*([W3.T] * 8))
    b1p = jnp.tile(b1, 8).reshape(1, 8 * H)
    b2p = jnp.tile(b2, 8).reshape(1, 8 * H)
    out = _mlp(e2, y2, w1p, wyp, b1p, w2p, b2p, w3p, b3)
    return out.reshape(B, 1)


# speedup vs baseline: 2.5763x; 1.1090x over previous
import functools

import jax
import jax.numpy as jnp
from jax import lax
from jax.scipy.linalg import block_diag
from jax.experimental import pallas as pl
from jax.experimental.pallas import tpu as pltpu
from jax.experimental.pallas import tpu_sc as plsc

B = 16384
D = 16
H = 32
NC, NS = 2, 16
NW = NC * NS
BPW = B // NW
L = 16
NVEC = BPW // L
EPW = BPW * D
GC = 128
NCHUNK = BPW // GC
G = B // 8
BG = 1024


@functools.cache
def _make_sc_gather():
    mesh = plsc.VectorSubcoreMesh(
        core_axis_name="c", subcore_axis_name="s", num_cores=NC, num_subcores=NS
    )

    @functools.partial(
        pl.kernel,
        out_type=jax.ShapeDtypeStruct((B * D,), jnp.float32),
        mesh=mesh,
        scratch_types=[
            pltpu.VMEM((BPW * 2,), jnp.float32),
            pltpu.VMEM((NCHUNK, GC), jnp.int32),
            pltpu.VMEM((BPW, D), jnp.float32),
            pltpu.VMEM((EPW,), jnp.float32),
            pltpu.SemaphoreType.DMA,
        ],
        compiler_params=pltpu.CompilerParams(
            use_tc_tiling_on_sc=False, needs_layout_passes=False
        ),
    )
    def sc_gather(x_hbm, table_hbm, out_hbm, x_v, idx_v, rows_v, out_v, sem):
        wid = lax.axis_index("s") * NC + lax.axis_index("c")
        base = wid * BPW
        pltpu.sync_copy(x_hbm.at[pl.ds(base * 2, BPW * 2)], x_v)

        iota = lax.iota(jnp.int32, L)
        iota2 = iota * 2

        @plsc.parallel_loop(0, NVEC, step=1)
        def extract(c):
            cf = plsc.load_gather(x_v, [c * (2 * L) + iota2])
            idx_v[c // (GC // L), pl.ds((c % (GC // L)) * L, L)] = (
                cf.astype(jnp.int32))

        copies = [
            pltpu.async_copy(
                table_hbm.at[idx_v.at[j]], rows_v.at[pl.ds(j * GC, GC)], sem
            )
            for j in range(NCHUNK)
        ]
        for c in copies:
            c.wait()

        @plsc.parallel_loop(0, NVEC, step=1)
        def flatten(c):
            for t in range(L):
                out_v[pl.ds((c * L + t) * D, D)] = rows_v[c * L + t]

        pltpu.sync_copy(out_v, out_hbm.at[pl.ds(wid * EPW, EPW)])

    return sc_gather


def _mlp_body(e_ref, y_ref, w1_ref, wy_ref, b1_ref, w2_ref, b2_ref,
              w3_ref, b3_ref, out_ref):
    h1 = lax.dot_general(e_ref[...], w1_ref[...], (((1,), (0,)), ((), ())),
                         preferred_element_type=jnp.float32)
    h1y = lax.dot_general(y_ref[...], wy_ref[...], (((1,), (0,)), ((), ())),
                          preferred_element_type=jnp.float32)
    h1 = jnp.maximum(h1 + h1y + b1_ref[...], 0.0)
    h2 = lax.dot_general(h1, w2_ref[...], (((1,), (0,)), ((), ())),
                         preferred_element_type=jnp.float32)
    h2 = jnp.maximum(h2 + b2_ref[...], 0.0)
    out = lax.dot_general(h2, w3_ref[...], (((1,), (0,)), ((), ())),
                          preferred_element_type=jnp.float32)
    out_ref[...] = out + b3_ref[0]


def _mlp(e2, y2, w1p, wyp, b1p, w2p, b2p, w3p, b3):
    full = lambda s: pl.BlockSpec(s, lambda i: (0, 0))
    return pl.pallas_call(
        _mlp_body,
        grid=(G // BG,),
        in_specs=[
            pl.BlockSpec((BG, 128), lambda i: (i, 0)),
            pl.BlockSpec((BG, 8), lambda i: (i, 0)),
            full((128, 256)),
            full((8, 256)),
            full((1, 256)),
            full((256, 256)),
            full((1, 256)),
            full((256, 8)),
            pl.BlockSpec(memory_space=pltpu.SMEM),
        ],
        out_specs=pl.BlockSpec((BG, 8), lambda i: (i, 0)),
        out_shape=jax.ShapeDtypeStruct((G, 8), jnp.float32),
    )(e2, y2, w1p, wyp, b1p, w2p, b2p, w3p, b3)


def kernel(x, embed, W1, b1, W2, b2, W3, b3):
    e_flat = _make_sc_gather()(x.reshape(B * 2), embed)
    e2 = e_flat.reshape(G, 8 * D)
    y2 = x[:, 1].reshape(G, 8)
    w1e_t = W1[:, 1:].T
    wy_t = W1[:, 0].reshape(1, H)
    w1p = block_diag(*([w1e_t] * 8))
    wyp = block_diag(*([wy_t] * 8))
    w2p = block_diag(*([W2.T] * 8))
    w3p = block_diag(*([W3.T] * 8))
    b1p = jnp.tile(b1, 8).reshape(1, 8 * H)
    b2p = jnp.tile(b2, 8).reshape(1, 8 * H)
    out = _mlp(e2, y2, w1p, wyp, b1p, w2p, b2p, w3p, b3)
    return out.reshape(B, 1)
